# baseline (speedup 1.0000x reference)
import functools

import jax
import jax.numpy as jnp
from jax import lax
from jax.experimental import pallas as pl
from jax.experimental.pallas import tpu as pltpu, tpu_sc as plsc

_B, _C, _L = 4096, 16, 512
_NUM = _B // 2
_ROWS = _B * _C
_BLK = 4096
_NBLK = _ROWS // _BLK
_W = _BLK // 32

_NC, _NS, _LANES = 2, 16, 16
_NW = _NC * _NS
_PER = _ROWS // _NW


def _sc_mask(flat_ids):
    mesh = plsc.VectorSubcoreMesh(
        core_axis_name="c", subcore_axis_name="s", num_cores=_NC, num_subcores=_NS
    )

    @functools.partial(
        pl.kernel,
        out_type=jax.ShapeDtypeStruct((_ROWS,), jnp.float32),
        mesh=mesh,
        scratch_types=[
            pltpu.VMEM((_NUM,), jnp.int32),
            pltpu.VMEM((_PER,), jnp.float32),
        ],
        compiler_params=pltpu.CompilerParams(needs_layout_passes=False),
    )
    def body(ids_hbm, mask_hbm, idv, mv):
        wid = lax.axis_index("s") * _NC + lax.axis_index("c")
        lo = wid * _PER
        pltpu.sync_copy(ids_hbm, idv)

        def fill(i, carry):
            mv[pl.ds(i * _LANES, _LANES)] = jnp.ones((_LANES,), jnp.float32)
            return carry

        lax.fori_loop(0, _PER // _LANES, fill, 0, unroll=8)

        def scat(j, carry):
            v = idv[pl.ds(j * _LANES, _LANES)]
            inr = (v >= lo) & (v < lo + _PER)
            safe = jnp.where(inr, v - lo, 0)
            plsc.store_scatter(
                mv, [safe], jnp.zeros((_LANES,), jnp.float32), mask=inr
            )
            return carry

        lax.fori_loop(0, _NUM // _LANES, scat, 0, unroll=8)
        pltpu.sync_copy(mv, mask_hbm.at[pl.ds(lo, _PER)])

    return body(flat_ids)


def _mute_body(m_ref, x_ref, o_ref):
    mute = m_ref[...][:, :, None]
    o_ref[...] = jnp.where(mute == 0.0, 0.0, x_ref[...])


def kernel(X):
    B, C, L = X.shape
    k = jax.random.key(42)
    k1, k2 = jax.random.split(k)
    channel = jax.random.randint(k1, (_NUM,), 0, C)
    indices = jax.random.randint(k2, (_NUM,), 0, B)
    flat_ids = indices * C + channel

    mask = _sc_mask(flat_ids).reshape(_ROWS // 32, 32)
    X3 = X.reshape(_ROWS // 32, 32, _L)
    out = pl.pallas_call(
        _mute_body,
        grid=(_NBLK,),
        in_specs=[
            pl.BlockSpec((_W, 32), lambda g: (g, 0)),
            pl.BlockSpec((_W, 32, _L), lambda g: (g, 0, 0)),
        ],
        out_specs=pl.BlockSpec((_W, 32, _L), lambda g: (g, 0, 0)),
        out_shape=jax.ShapeDtypeStruct((_ROWS // 32, 32, _L), X.dtype),
    )(mask, X3)
    return out.reshape(B, C, L), indices


# speedup vs baseline: 1.2062x; 1.2062x over previous
import jax
import jax.numpy as jnp
from jax.experimental import pallas as pl

_B, _C, _L = 4096, 16, 512
_NUM = _B // 2
_ROWS = _B * _C
_BLK = 4096
_NBLK = _ROWS // _BLK
_W = _BLK // 32
_IDS_R, _IDS_C = 16, 128


def _body(ids_ref, x_ref, o_ref):
    g = pl.program_id(0)
    wcol = g * _W + jax.lax.broadcasted_iota(jnp.int32, (_W, _IDS_C), 0)
    acc = jnp.zeros((_W, _IDS_C), jnp.int32)
    for c in range(_IDS_R):
        idv = ids_ref[c, :]
        idw = (idv >> 5)[None, :]
        bit = (1 << (idv & 31))[None, :]
        acc = acc | jnp.where(wcol == idw, bit, 0)
    r = acc
    s = _IDS_C // 2
    while s >= 1:
        r = jax.lax.slice_in_dim(r, 0, s, axis=1) | jax.lax.slice_in_dim(r, s, 2 * s, axis=1)
        s //= 2
    bits = jax.lax.broadcasted_iota(jnp.int32, (_W, 32, 1), 1)
    mute = (r.reshape(_W, 1, 1) >> bits) & 1
    o_ref[...] = jnp.where(mute != 0, 0.0, x_ref[...])


def kernel(X):
    B, C, L = X.shape
    k = jax.random.key(42)
    k1, k2 = jax.random.split(k)
    channel = jax.random.randint(k1, (_NUM,), 0, C)
    indices = jax.random.randint(k2, (_NUM,), 0, B)
    flat_ids = (indices * C + channel).reshape(_IDS_R, _IDS_C)
    X3 = X.reshape(_ROWS // 32, 32, _L)
    out = pl.pallas_call(
        _body,
        grid=(_NBLK,),
        in_specs=[
            pl.BlockSpec((_IDS_R, _IDS_C), lambda g: (0, 0)),
            pl.BlockSpec((_W, 32, _L), lambda g: (g, 0, 0)),
        ],
        out_specs=pl.BlockSpec((_W, 32, _L), lambda g: (g, 0, 0)),
        out_shape=jax.ShapeDtypeStruct((_ROWS // 32, 32, _L), X.dtype),
    )(flat_ids, X3)
    return out.reshape(B, C, L), indices


# speedup vs baseline: 1.2307x; 1.0204x over previous
import jax
import jax.numpy as jnp
from jax.experimental import pallas as pl

_B, _C, _L = 4096, 16, 512
_NUM = _B // 2
_ROWS = _B * _C
_BLK = 6144
_NBLK = -(-_ROWS // _BLK)
_W = _BLK // 32
_IDS_R, _IDS_C = 16, 128


def _body(ids_ref, x_ref, o_ref):
    g = pl.program_id(0)
    wcol = g * _W + jax.lax.broadcasted_iota(jnp.int32, (_W, _IDS_C), 0)
    acc = jnp.zeros((_W, _IDS_C), jnp.int32)
    for c in range(_IDS_R):
        idv = ids_ref[c, :]
        idw = (idv >> 5)[None, :]
        bit = (1 << (idv & 31))[None, :]
        acc = acc | jnp.where(wcol == idw, bit, 0)
    r = acc
    s = _IDS_C // 2
    while s >= 1:
        r = jax.lax.slice_in_dim(r, 0, s, axis=1) | jax.lax.slice_in_dim(r, s, 2 * s, axis=1)
        s //= 2
    bits = jax.lax.broadcasted_iota(jnp.int32, (_W, 32, 1), 1)
    mute = (r.reshape(_W, 1, 1) >> bits) & 1
    o_ref[...] = jnp.where(mute != 0, 0.0, x_ref[...])


def kernel(X):
    B, C, L = X.shape
    k = jax.random.key(42)
    k1, k2 = jax.random.split(k)
    channel = jax.random.randint(k1, (_NUM,), 0, C)
    indices = jax.random.randint(k2, (_NUM,), 0, B)
    flat_ids = (indices * C + channel).reshape(_IDS_R, _IDS_C)
    X3 = X.reshape(_ROWS // 32, 32, _L)
    out = pl.pallas_call(
        _body,
        grid=(_NBLK,),
        in_specs=[
            pl.BlockSpec((_IDS_R, _IDS_C), lambda g: (0, 0)),
            pl.BlockSpec((_W, 32, _L), lambda g: (g, 0, 0)),
        ],
        out_specs=pl.BlockSpec((_W, 32, _L), lambda g: (g, 0, 0)),
        out_shape=jax.ShapeDtypeStruct((_ROWS // 32, 32, _L), X.dtype),
    )(flat_ids, X3)
    return out.reshape(B, C, L), indices


# speedup vs baseline: 1.2353x; 1.0037x over previous
import jax
import jax.numpy as jnp
from jax.experimental import pallas as pl
from jax.experimental.pallas import tpu as pltpu

_B, _C, _L = 4096, 16, 512
_NUM = _B // 2
_ROWS = _B * _C
_BLK = 7680
_NBLK = -(-_ROWS // _BLK)
_W = _BLK // 32
_IDS_R, _IDS_C = 16, 128


def _body(ids_ref, x_ref, o_ref):
    g = pl.program_id(0)
    wcol = g * _W + jax.lax.broadcasted_iota(jnp.int32, (_W, _IDS_C), 0)
    acc = jnp.zeros((_W, _IDS_C), jnp.int32)
    for c in range(_IDS_R):
        idv = ids_ref[c, :]
        idw = (idv >> 5)[None, :]
        bit = (1 << (idv & 31))[None, :]
        acc = acc | jnp.where(wcol == idw, bit, 0)
    r = acc
    s = _IDS_C // 2
    while s >= 1:
        r = jax.lax.slice_in_dim(r, 0, s, axis=1) | jax.lax.slice_in_dim(r, s, 2 * s, axis=1)
        s //= 2
    bits = jax.lax.broadcasted_iota(jnp.int32, (_W, 32, 1), 1)
    mute = (r.reshape(_W, 1, 1) >> bits) & 1
    o_ref[...] = jnp.where(mute != 0, 0.0, x_ref[...])


def kernel(X):
    B, C, L = X.shape
    k = jax.random.key(42)
    k1, k2 = jax.random.split(k)
    channel = jax.random.randint(k1, (_NUM,), 0, C)
    indices = jax.random.randint(k2, (_NUM,), 0, B)
    flat_ids = (indices * C + channel).reshape(_IDS_R, _IDS_C)
    X3 = X.reshape(_ROWS // 32, 32, _L)
    out = pl.pallas_call(
        _body,
        grid=(_NBLK,),
        in_specs=[
            pl.BlockSpec((_IDS_R, _IDS_C), lambda g: (0, 0)),
            pl.BlockSpec((_W, 32, _L), lambda g: (g, 0, 0)),
        ],
        out_specs=pl.BlockSpec((_W, 32, _L), lambda g: (g, 0, 0)),
        out_shape=jax.ShapeDtypeStruct((_ROWS // 32, 32, _L), X.dtype),
        compiler_params=pltpu.CompilerParams(vmem_limit_bytes=63 * 1024 * 1024),
    )(flat_ids, X3)
    return out.reshape(B, C, L), indices
